# baseline (speedup 1.0000x reference)
import jax
import jax.numpy as jnp
from jax import lax
from jax.experimental import pallas as pl
from jax.experimental.pallas import tpu as pltpu
from jax.experimental.pallas import tpu_sc as plsc

N = 100000
E = 3200000
NPAD = 100096
SLICE = NPAD // 16
EB = 128
CB = 8
CHUNK = EB * CB
NCHUNKS = E // CHUNK
NW = 32
NCH = NCHUNKS // NW
EXTRA = NCHUNKS - NCH * NW
D = 16
HSLICE = (NPAD // 2) // 16
R = NPAD // 32

_SC_PARAMS = pltpu.CompilerParams(use_tc_tiling_on_sc=False)


def _mesh():
    return plsc.VectorSubcoreMesh(
        core_axis_name="c", subcore_axis_name="s", num_cores=2, num_subcores=16
    )


def _worker_ids():
    c = lax.axis_index("c")
    s = lax.axis_index("s")
    wid = c * 16 + s
    nch = jnp.where(wid < EXTRA, NCH + 1, NCH).astype(jnp.int32)
    start = (wid * NCH + jnp.minimum(wid, EXTRA)).astype(jnp.int32)
    return c, s, nch, start


def _sc_deg(dst2, ew2):
    def run(refs):
        dst_ref, ew_ref, out_ref = refs

        @pl.core_map(
            _mesh(),
            compiler_params=_SC_PARAMS,
            scratch_shapes=[
                pltpu.VMEM((2, CB, EB), jnp.int32),
                pltpu.VMEM((2, CB, EB), jnp.float32),
                pltpu.VMEM((SLICE,), jnp.float32),
                pltpu.VMEM_SHARED((NPAD,), jnp.float32),
                pltpu.SemaphoreType.DMA,
                pltpu.SemaphoreType.DMA,
            ],
        )
        def _(dstb, ewb, obuf, acc, lsem, ssem):
            c, s, nch, start = _worker_ids()
            r0 = s * SLICE
            zv = jnp.zeros((16,), jnp.float32)

            def zrow(r, zc):
                obuf[pl.ds(r * 16, 16)] = zv
                return zc

            lax.fori_loop(0, SLICE // 16, zrow, 0, unroll=8)
            pltpu.sync_copy(obuf, acc.at[pl.ds(r0, SLICE)])
            plsc.subcore_barrier()

            def load_chunk(ci2, cs2):
                blk = (start + ci2) * CB
                pltpu.async_copy(
                    dst_ref.at[pl.ds(blk, CB)], dstb.at[cs2], lsem)
                pltpu.async_copy(
                    ew_ref.at[pl.ds(blk, CB)], ewb.at[cs2], lsem)

            def drain_chunk(ci2, cs2):
                blk = (start + ci2) * CB
                pltpu.make_async_copy(
                    dst_ref.at[pl.ds(blk, CB)], dstb.at[cs2], lsem).wait()
                pltpu.make_async_copy(
                    ew_ref.at[pl.ds(blk, CB)], ewb.at[cs2], lsem).wait()

            def scatter_wait_chunk(cs2):
                for j in range(CB):
                    pltpu.make_async_copy(
                        ewb.at[cs2, j], acc.at[dstb.at[cs2, j]], ssem).wait()

            load_chunk(0, 0)
            drain_chunk(0, 0)

            def chunk(ci, carry):
                cs = lax.rem(ci, 2)
                ncs = lax.rem(ci + 1, 2)

                @pl.when(ci > 0)
                def _():
                    scatter_wait_chunk(ncs)

                @pl.when(ci + 1 < nch)
                def _():
                    load_chunk(ci + 1, ncs)

                for j in range(CB):
                    pltpu.async_copy(
                        ewb.at[cs, j], acc.at[dstb.at[cs, j]], ssem, add=True)

                @pl.when(ci + 1 < nch)
                def _():
                    drain_chunk(ci + 1, ncs)

                return carry

            lax.fori_loop(0, nch, chunk, 0)
            fs = lax.rem(nch - 1, 2)
            for j in range(CB):
                pltpu.make_async_copy(
                    ewb.at[fs, j], acc.at[dstb.at[fs, j]], ssem).wait()
            plsc.subcore_barrier()
            pltpu.sync_copy(acc.at[pl.ds(r0, SLICE)], obuf)
            pltpu.sync_copy(obuf, out_ref.at[c, pl.ds(r0, SLICE)])

    out = pl.run_state(run)(
        (dst2, ew2, jnp.zeros((2, NPAD), jnp.float32)))
    return out[2]


def _sc_agg(src2, dst2, ew2, tab):
    def run(refs):
        src_ref, dst_ref, ew_ref, tab_ref, out_ref = refs

        @pl.core_map(
            _mesh(),
            compiler_params=_SC_PARAMS,
            scratch_shapes=[
                pltpu.VMEM((2, CB, EB), jnp.int32),
                pltpu.VMEM((2, CB, EB), jnp.int32),
                pltpu.VMEM((2, CB, EB), jnp.float32),
                pltpu.VMEM((4, EB), jnp.int32),
                pltpu.VMEM((4, EB, D), jnp.float32),
                pltpu.VMEM((4, EB, D), jnp.float32),
                pltpu.VMEM((HSLICE, D), jnp.float32),
                pltpu.VMEM_SHARED((NPAD // 2, D), jnp.float32),
                pltpu.SemaphoreType.DMA,
                pltpu.SemaphoreType.DMA,
                pltpu.SemaphoreType.DMA,
                pltpu.SemaphoreType.DMA,
                pltpu.SemaphoreType.DMA,
                pltpu.SemaphoreType.DMA,
                pltpu.SemaphoreType.DMA,
                pltpu.SemaphoreType.DMA,
                pltpu.SemaphoreType.DMA,
            ],
        )
        def _(srcb, dstb, ewb, idxb, rbuf, sbuf, obuf, acc,
              gsem0, gsem1, gsem2, gsem3, ssem0, ssem1, ssem2, ssem3, lsem):
            c, s, nch, start = _worker_ids()
            r0 = s * HSLICE
            zv = jnp.zeros((16,), jnp.float32)

            def zrow(r, zc):
                obuf[r, :] = zv
                return zc

            lax.fori_loop(0, HSLICE, zrow, 0, unroll=8)
            pltpu.sync_copy(obuf, acc.at[pl.ds(r0, HSLICE)])
            plsc.subcore_barrier()

            lt8 = lax.iota(jnp.int32, 16) < 8
            gsem = (gsem0, gsem1, gsem2, gsem3)
            ssem = (ssem0, ssem1, ssem2, ssem3)

            def gather(cs, j, p):
                pltpu.async_copy(
                    tab_ref.at[srcb.at[cs, j]], rbuf.at[p], gsem[p])

            def gather_wait(cs, j, p):
                pltpu.make_async_copy(
                    tab_ref.at[srcb.at[cs, j]], rbuf.at[p], gsem[p]).wait()

            def scatter(p):
                pltpu.async_copy(
                    sbuf.at[p], acc.at[idxb.at[p]], ssem[p], add=True)

            def scatter_wait(p):
                pltpu.make_async_copy(
                    sbuf.at[p], acc.at[idxb.at[p]], ssem[p]).wait()

            def load_chunk(ci2, cs2):
                blk = (start + ci2) * CB
                pltpu.async_copy(
                    src_ref.at[pl.ds(blk, CB)], srcb.at[cs2], lsem)
                pltpu.async_copy(
                    dst_ref.at[pl.ds(blk, CB)], dstb.at[cs2], lsem)
                pltpu.async_copy(
                    ew_ref.at[pl.ds(blk, CB)], ewb.at[cs2], lsem)

            def drain_chunk(ci2, cs2):
                blk = (start + ci2) * CB
                pltpu.make_async_copy(
                    src_ref.at[pl.ds(blk, CB)], srcb.at[cs2], lsem).wait()
                pltpu.make_async_copy(
                    dst_ref.at[pl.ds(blk, CB)], dstb.at[cs2], lsem).wait()
                pltpu.make_async_copy(
                    ew_ref.at[pl.ds(blk, CB)], ewb.at[cs2], lsem).wait()

            def compute(cs, j, p):
                def grp(g, gc):
                    base = g * 16
                    dstv = dstb[cs, j, pl.ds(base, 16)]
                    ewv = ewb[cs, j, pl.ds(base, 16)]
                    idxb[p, pl.ds(base, 16)] = lax.shift_right_logical(
                        dstv, 1)
                    parf = lax.convert_element_type(
                        lax.bitwise_and(dstv, 1), jnp.float32)
                    for k in range(16):
                        w = jnp.where(lt8, 1.0 - parf[k], parf[k])
                        sbuf[p, base + k, :] = (rbuf[p, base + k, :]
                                                * (ewv[k] * w))
                    return gc

                lax.fori_loop(0, EB // 16, grp, 0)

            load_chunk(0, 0)
            drain_chunk(0, 0)
            gather(0, 0, 0)
            gather(0, 1, 1)
            gather(0, 2, 2)

            def chunk(ci, carry):
                cs = lax.rem(ci, 2)
                ncs = lax.rem(ci + 1, 2)

                @pl.when(ci + 1 < nch)
                def _():
                    load_chunk(ci + 1, ncs)

                for j in range(CB):
                    p = j % 4
                    gather_wait(cs, j, p)

                    nj = j + 3
                    if nj < CB:
                        gather(cs, nj, nj % 4)
                    else:
                        if nj == CB:
                            @pl.when(ci + 1 < nch)
                            def _():
                                drain_chunk(ci + 1, ncs)

                        @pl.when(ci + 1 < nch)
                        def _():
                            gather(ncs, nj - CB, nj % 4)

                    if j >= 4:
                        scatter_wait(p)
                    else:
                        @pl.when(ci > 0)
                        def _():
                            scatter_wait(p)

                    compute(cs, j, p)
                    scatter(p)

                return carry

            lax.fori_loop(0, nch, chunk, 0)
            scatter_wait(0)
            scatter_wait(1)
            scatter_wait(2)
            scatter_wait(3)
            plsc.subcore_barrier()
            pltpu.sync_copy(acc.at[pl.ds(r0, HSLICE)], obuf)
            pltpu.sync_copy(obuf, out_ref.at[c, pl.ds(r0, HSLICE)])

    out = pl.run_state(run)(
        (src2, dst2, ew2, tab,
         jnp.zeros((2, NPAD // 2, D), jnp.float32)))
    return out[4]


def _prep1_body(x_ref, w_ref, degt_ref, hs_ref, sl_ref, dis_ref, dinv_ref):
    deg = degt_ref[:, 0:1] + degt_ref[:, 1:2] + 1.0
    dis = lax.rsqrt(deg)
    dinv = 1.0 / deg
    h8 = jnp.dot(x_ref[...], w_ref[...], preferred_element_type=jnp.float32)
    hd = h8 * dis
    hs_ref[...] = jnp.concatenate([hd, hd], axis=1)
    sl_ref[...] = h8 * dinv
    dis_ref[...] = dis
    dinv_ref[...] = dinv


def _prep2_body(a0_ref, a1_ref, dis_ref, dinv_ref, sl1_ref, b1_ref,
                xemb_ref, hs2_ref, sl2_ref):
    dis = dis_ref[...]
    xemb = (a0_ref[0] + a1_ref[0]) * dis + sl1_ref[...] + b1_ref[...]
    h2r = jnp.maximum(xemb, 0.0)
    h2d = h2r * dis
    xemb_ref[...] = xemb
    hs2_ref[...] = jnp.concatenate([h2d, h2d], axis=1)
    sl2_ref[...] = h2r * dinv_ref[...]


def _final_body(a0_ref, a1_ref, dis_ref, sl2_ref, w2_ref, b2_ref, out_ref):
    h = (a0_ref[0] + a1_ref[0]) * dis_ref[...] + sl2_ref[...]
    out_ref[...] = jnp.dot(h, w2_ref[...],
                           preferred_element_type=jnp.float32) + b2_ref[...]


def _col(i):
    return (i, 0)


def kernel(x, edge_index, edge_attr, W1, b1, W2, b2):
    src2 = edge_index[0].reshape(E // EB, EB)
    dst2 = edge_index[1].reshape(E // EB, EB)
    ew2 = edge_attr.reshape(E // EB, EB)
    degp = _sc_deg(dst2, ew2)
    degt = degp.T

    grid = (NPAD // R,)

    hs1, sl1, dis, dinv = pl.pallas_call(
        _prep1_body,
        grid=grid,
        in_specs=[
            pl.BlockSpec((R, 16), _col),
            pl.BlockSpec((16, 8), lambda i: (0, 0)),
            pl.BlockSpec((R, 2), _col),
        ],
        out_specs=[
            pl.BlockSpec((R, 16), _col),
            pl.BlockSpec((R, 8), _col),
            pl.BlockSpec((R, 1), _col),
            pl.BlockSpec((R, 1), _col),
        ],
        out_shape=[
            jax.ShapeDtypeStruct((N, 16), jnp.float32),
            jax.ShapeDtypeStruct((N, 8), jnp.float32),
            jax.ShapeDtypeStruct((N, 1), jnp.float32),
            jax.ShapeDtypeStruct((N, 1), jnp.float32),
        ],
    )(x, W1, degt)

    aggp1 = _sc_agg(src2, dst2, ew2, hs1).reshape(2, NPAD, 8)

    xemb, hs2, sl2 = pl.pallas_call(
        _prep2_body,
        grid=grid,
        in_specs=[
            pl.BlockSpec((1, R, 8), lambda i: (0, i, 0)),
            pl.BlockSpec((1, R, 8), lambda i: (1, i, 0)),
            pl.BlockSpec((R, 1), _col),
            pl.BlockSpec((R, 1), _col),
            pl.BlockSpec((R, 8), _col),
            pl.BlockSpec((1, 8), lambda i: (0, 0)),
        ],
        out_specs=[
            pl.BlockSpec((R, 8), _col),
            pl.BlockSpec((R, 16), _col),
            pl.BlockSpec((R, 8), _col),
        ],
        out_shape=[
            jax.ShapeDtypeStruct((N, 8), jnp.float32),
            jax.ShapeDtypeStruct((N, 16), jnp.float32),
            jax.ShapeDtypeStruct((N, 8), jnp.float32),
        ],
    )(aggp1, aggp1, dis, dinv, sl1, b1.reshape(1, 8))

    aggp2 = _sc_agg(src2, dst2, ew2, hs2).reshape(2, NPAD, 8)

    outp = pl.pallas_call(
        _final_body,
        grid=grid,
        in_specs=[
            pl.BlockSpec((1, R, 8), lambda i: (0, i, 0)),
            pl.BlockSpec((1, R, 8), lambda i: (1, i, 0)),
            pl.BlockSpec((R, 1), _col),
            pl.BlockSpec((R, 8), _col),
            pl.BlockSpec((8, 2), lambda i: (0, 0)),
            pl.BlockSpec((1, 2), lambda i: (0, 0)),
        ],
        out_specs=pl.BlockSpec((R, 2), _col),
        out_shape=jax.ShapeDtypeStruct((N, 2), jnp.float32),
    )(aggp2, aggp2, dis, sl2, W2, b2.reshape(1, 2))

    return (outp, xemb)


# speedup vs baseline: 1.0058x; 1.0058x over previous
import jax
import jax.numpy as jnp
from jax import lax
from jax.experimental import pallas as pl
from jax.experimental.pallas import tpu as pltpu
from jax.experimental.pallas import tpu_sc as plsc

N = 100000
E = 3200000
NPAD = 100096
SLICE = NPAD // 16
EB = 128
CB = 8
CHUNK = EB * CB
NCHUNKS = E // CHUNK
NW = 32
NCH = NCHUNKS // NW
EXTRA = NCHUNKS - NCH * NW
D = 16
HSLICE = (NPAD // 2) // 16
R = NPAD // 32

_SC_PARAMS = pltpu.CompilerParams(use_tc_tiling_on_sc=False)


def _mesh():
    return plsc.VectorSubcoreMesh(
        core_axis_name="c", subcore_axis_name="s", num_cores=2, num_subcores=16
    )


def _worker_ids():
    c = lax.axis_index("c")
    s = lax.axis_index("s")
    wid = c * 16 + s
    nch = jnp.where(wid < EXTRA, NCH + 1, NCH).astype(jnp.int32)
    start = (wid * NCH + jnp.minimum(wid, EXTRA)).astype(jnp.int32)
    return c, s, nch, start


def _sc_deg(dst2, ew2, zero1):
    def run(refs):
        dst_ref, ew_ref, zero_ref, out_ref = refs

        @pl.core_map(
            _mesh(),
            compiler_params=_SC_PARAMS,
            scratch_shapes=[
                pltpu.VMEM((2, CB, EB), jnp.int32),
                pltpu.VMEM((2, CB, EB), jnp.float32),
                pltpu.VMEM((SLICE,), jnp.float32),
                pltpu.VMEM_SHARED((NPAD,), jnp.float32),
                pltpu.SemaphoreType.DMA,
                pltpu.SemaphoreType.DMA,
            ],
        )
        def _(dstb, ewb, obuf, acc, lsem, ssem):
            c, s, nch, start = _worker_ids()
            r0 = s * SLICE
            pltpu.sync_copy(zero_ref.at[pl.ds(r0, SLICE)], obuf)
            pltpu.sync_copy(obuf, acc.at[pl.ds(r0, SLICE)])
            plsc.subcore_barrier()

            def load_chunk(ci2, cs2):
                blk = (start + ci2) * CB
                pltpu.async_copy(
                    dst_ref.at[pl.ds(blk, CB)], dstb.at[cs2], lsem)
                pltpu.async_copy(
                    ew_ref.at[pl.ds(blk, CB)], ewb.at[cs2], lsem)

            def drain_chunk(ci2, cs2):
                blk = (start + ci2) * CB
                pltpu.make_async_copy(
                    dst_ref.at[pl.ds(blk, CB)], dstb.at[cs2], lsem).wait()
                pltpu.make_async_copy(
                    ew_ref.at[pl.ds(blk, CB)], ewb.at[cs2], lsem).wait()

            def scatter_wait_chunk(cs2):
                for j in range(CB):
                    pltpu.make_async_copy(
                        ewb.at[cs2, j], acc.at[dstb.at[cs2, j]], ssem).wait()

            load_chunk(0, 0)
            drain_chunk(0, 0)

            def chunk(ci, carry):
                cs = lax.rem(ci, 2)
                ncs = lax.rem(ci + 1, 2)

                @pl.when(ci > 0)
                def _():
                    scatter_wait_chunk(ncs)

                @pl.when(ci + 1 < nch)
                def _():
                    load_chunk(ci + 1, ncs)

                for j in range(CB):
                    pltpu.async_copy(
                        ewb.at[cs, j], acc.at[dstb.at[cs, j]], ssem, add=True)

                @pl.when(ci + 1 < nch)
                def _():
                    drain_chunk(ci + 1, ncs)

                return carry

            lax.fori_loop(0, nch, chunk, 0)
            fs = lax.rem(nch - 1, 2)
            for j in range(CB):
                pltpu.make_async_copy(
                    ewb.at[fs, j], acc.at[dstb.at[fs, j]], ssem).wait()
            plsc.subcore_barrier()
            pltpu.sync_copy(acc.at[pl.ds(r0, SLICE)], obuf)
            pltpu.sync_copy(obuf, out_ref.at[c, pl.ds(r0, SLICE)])

    out = pl.run_state(run)(
        (dst2, ew2, zero1, jnp.zeros((2, NPAD), jnp.float32)))
    return out[3]


def _sc_agg(src2, dst2, ew2, tab, zero2):
    def run(refs):
        src_ref, dst_ref, ew_ref, tab_ref, zero_ref, out_ref = refs

        @pl.core_map(
            _mesh(),
            compiler_params=_SC_PARAMS,
            scratch_shapes=[
                pltpu.VMEM((2, CB, EB), jnp.int32),
                pltpu.VMEM((2, CB, EB), jnp.int32),
                pltpu.VMEM((2, CB, EB), jnp.float32),
                pltpu.VMEM((4, EB), jnp.int32),
                pltpu.VMEM((4, EB, D), jnp.float32),
                pltpu.VMEM((4, EB, D), jnp.float32),
                pltpu.VMEM((HSLICE, D), jnp.float32),
                pltpu.VMEM_SHARED((NPAD // 2, D), jnp.float32),
                pltpu.SemaphoreType.DMA,
                pltpu.SemaphoreType.DMA,
                pltpu.SemaphoreType.DMA,
                pltpu.SemaphoreType.DMA,
                pltpu.SemaphoreType.DMA,
                pltpu.SemaphoreType.DMA,
                pltpu.SemaphoreType.DMA,
                pltpu.SemaphoreType.DMA,
                pltpu.SemaphoreType.DMA,
            ],
        )
        def _(srcb, dstb, ewb, idxb, rbuf, sbuf, obuf, acc,
              gsem0, gsem1, gsem2, gsem3, ssem0, ssem1, ssem2, ssem3, lsem):
            c, s, nch, start = _worker_ids()
            r0 = s * HSLICE
            pltpu.sync_copy(zero_ref.at[pl.ds(r0, HSLICE)], obuf)
            pltpu.sync_copy(obuf, acc.at[pl.ds(r0, HSLICE)])
            plsc.subcore_barrier()

            lt8 = lax.iota(jnp.int32, 16) < 8
            gsem = (gsem0, gsem1, gsem2, gsem3)
            ssem = (ssem0, ssem1, ssem2, ssem3)

            def gather(cs, j, p):
                pltpu.async_copy(
                    tab_ref.at[srcb.at[cs, j]], rbuf.at[p], gsem[p])

            def gather_wait(cs, j, p):
                pltpu.make_async_copy(
                    tab_ref.at[srcb.at[cs, j]], rbuf.at[p], gsem[p]).wait()

            def scatter(p):
                pltpu.async_copy(
                    sbuf.at[p], acc.at[idxb.at[p]], ssem[p], add=True)

            def scatter_wait(p):
                pltpu.make_async_copy(
                    sbuf.at[p], acc.at[idxb.at[p]], ssem[p]).wait()

            def load_chunk(ci2, cs2):
                blk = (start + ci2) * CB
                pltpu.async_copy(
                    src_ref.at[pl.ds(blk, CB)], srcb.at[cs2], lsem)
                pltpu.async_copy(
                    dst_ref.at[pl.ds(blk, CB)], dstb.at[cs2], lsem)
                pltpu.async_copy(
                    ew_ref.at[pl.ds(blk, CB)], ewb.at[cs2], lsem)

            def drain_chunk(ci2, cs2):
                blk = (start + ci2) * CB
                pltpu.make_async_copy(
                    src_ref.at[pl.ds(blk, CB)], srcb.at[cs2], lsem).wait()
                pltpu.make_async_copy(
                    dst_ref.at[pl.ds(blk, CB)], dstb.at[cs2], lsem).wait()
                pltpu.make_async_copy(
                    ew_ref.at[pl.ds(blk, CB)], ewb.at[cs2], lsem).wait()

            def compute(cs, j, p):
                def grp(g, gc):
                    base = g * 16
                    dstv = dstb[cs, j, pl.ds(base, 16)]
                    ewv = ewb[cs, j, pl.ds(base, 16)]
                    idxb[p, pl.ds(base, 16)] = lax.shift_right_logical(
                        dstv, 1)
                    parf = lax.convert_element_type(
                        lax.bitwise_and(dstv, 1), jnp.float32)
                    for k in range(16):
                        w = jnp.where(lt8, 1.0 - parf[k], parf[k])
                        sbuf[p, base + k, :] = (rbuf[p, base + k, :]
                                                * (ewv[k] * w))
                    return gc

                lax.fori_loop(0, EB // 16, grp, 0)

            load_chunk(0, 0)
            drain_chunk(0, 0)
            gather(0, 0, 0)
            gather(0, 1, 1)
            gather(0, 2, 2)

            def chunk(ci, carry):
                cs = lax.rem(ci, 2)
                ncs = lax.rem(ci + 1, 2)

                @pl.when(ci + 1 < nch)
                def _():
                    load_chunk(ci + 1, ncs)

                for j in range(CB):
                    p = j % 4
                    gather_wait(cs, j, p)

                    nj = j + 3
                    if nj < CB:
                        gather(cs, nj, nj % 4)
                    else:
                        if nj == CB:
                            @pl.when(ci + 1 < nch)
                            def _():
                                drain_chunk(ci + 1, ncs)

                        @pl.when(ci + 1 < nch)
                        def _():
                            gather(ncs, nj - CB, nj % 4)

                    if j >= 4:
                        scatter_wait(p)
                    else:
                        @pl.when(ci > 0)
                        def _():
                            scatter_wait(p)

                    compute(cs, j, p)
                    scatter(p)

                return carry

            lax.fori_loop(0, nch, chunk, 0)
            scatter_wait(0)
            scatter_wait(1)
            scatter_wait(2)
            scatter_wait(3)
            plsc.subcore_barrier()
            pltpu.sync_copy(acc.at[pl.ds(r0, HSLICE)], obuf)
            pltpu.sync_copy(obuf, out_ref.at[c, pl.ds(r0, HSLICE)])

    out = pl.run_state(run)(
        (src2, dst2, ew2, tab, zero2,
         jnp.zeros((2, NPAD // 2, D), jnp.float32)))
    return out[5]


def _prep1_body(x_ref, w_ref, degt_ref, hs_ref, sl_ref, dis_ref, dinv_ref):
    deg = degt_ref[:, 0:1] + degt_ref[:, 1:2] + 1.0
    dis = lax.rsqrt(deg)
    dinv = 1.0 / deg
    h8 = jnp.dot(x_ref[...], w_ref[...], preferred_element_type=jnp.float32)
    hd = h8 * dis
    hs_ref[...] = jnp.concatenate([hd, hd], axis=1)
    sl_ref[...] = h8 * dinv
    dis_ref[...] = dis
    dinv_ref[...] = dinv


def _prep2_body(a0_ref, a1_ref, dis_ref, dinv_ref, sl1_ref, b1_ref,
                xemb_ref, hs2_ref, sl2_ref):
    dis = dis_ref[...]
    xemb = (a0_ref[0] + a1_ref[0]) * dis + sl1_ref[...] + b1_ref[...]
    h2r = jnp.maximum(xemb, 0.0)
    h2d = h2r * dis
    xemb_ref[...] = xemb
    hs2_ref[...] = jnp.concatenate([h2d, h2d], axis=1)
    sl2_ref[...] = h2r * dinv_ref[...]


def _final_body(a0_ref, a1_ref, dis_ref, sl2_ref, w2_ref, b2_ref, out_ref):
    h = (a0_ref[0] + a1_ref[0]) * dis_ref[...] + sl2_ref[...]
    out_ref[...] = jnp.dot(h, w2_ref[...],
                           preferred_element_type=jnp.float32) + b2_ref[...]


def _col(i):
    return (i, 0)


def kernel(x, edge_index, edge_attr, W1, b1, W2, b2):
    src2 = edge_index[0].reshape(E // EB, EB)
    dst2 = edge_index[1].reshape(E // EB, EB)
    ew2 = edge_attr.reshape(E // EB, EB)
    z1 = jnp.zeros((NPAD,), jnp.float32)
    z16 = jnp.zeros((NPAD // 2, D), jnp.float32)

    degp = _sc_deg(dst2, ew2, z1)
    degt = degp.T

    grid = (NPAD // R,)

    hs1, sl1, dis, dinv = pl.pallas_call(
        _prep1_body,
        grid=grid,
        in_specs=[
            pl.BlockSpec((R, 16), _col),
            pl.BlockSpec((16, 8), lambda i: (0, 0)),
            pl.BlockSpec((R, 2), _col),
        ],
        out_specs=[
            pl.BlockSpec((R, 16), _col),
            pl.BlockSpec((R, 8), _col),
            pl.BlockSpec((R, 1), _col),
            pl.BlockSpec((R, 1), _col),
        ],
        out_shape=[
            jax.ShapeDtypeStruct((N, 16), jnp.float32),
            jax.ShapeDtypeStruct((N, 8), jnp.float32),
            jax.ShapeDtypeStruct((N, 1), jnp.float32),
            jax.ShapeDtypeStruct((N, 1), jnp.float32),
        ],
    )(x, W1, degt)

    aggp1 = _sc_agg(src2, dst2, ew2, hs1, z16).reshape(2, NPAD, 8)

    xemb, hs2, sl2 = pl.pallas_call(
        _prep2_body,
        grid=grid,
        in_specs=[
            pl.BlockSpec((1, R, 8), lambda i: (0, i, 0)),
            pl.BlockSpec((1, R, 8), lambda i: (1, i, 0)),
            pl.BlockSpec((R, 1), _col),
            pl.BlockSpec((R, 1), _col),
            pl.BlockSpec((R, 8), _col),
            pl.BlockSpec((1, 8), lambda i: (0, 0)),
        ],
        out_specs=[
            pl.BlockSpec((R, 8), _col),
            pl.BlockSpec((R, 16), _col),
            pl.BlockSpec((R, 8), _col),
        ],
        out_shape=[
            jax.ShapeDtypeStruct((N, 8), jnp.float32),
            jax.ShapeDtypeStruct((N, 16), jnp.float32),
            jax.ShapeDtypeStruct((N, 8), jnp.float32),
        ],
    )(aggp1, aggp1, dis, dinv, sl1, b1.reshape(1, 8))

    aggp2 = _sc_agg(src2, dst2, ew2, hs2, z16).reshape(2, NPAD, 8)

    outp = pl.pallas_call(
        _final_body,
        grid=grid,
        in_specs=[
            pl.BlockSpec((1, R, 8), lambda i: (0, i, 0)),
            pl.BlockSpec((1, R, 8), lambda i: (1, i, 0)),
            pl.BlockSpec((R, 1), _col),
            pl.BlockSpec((R, 8), _col),
            pl.BlockSpec((8, 2), lambda i: (0, 0)),
            pl.BlockSpec((1, 2), lambda i: (0, 0)),
        ],
        out_specs=pl.BlockSpec((R, 2), _col),
        out_shape=jax.ShapeDtypeStruct((N, 2), jnp.float32),
    )(aggp2, aggp2, dis, sl2, W2, b2.reshape(1, 2))

    return (outp, xemb)
